# Initial kernel scaffold; baseline (speedup 1.0000x reference)
import jax
import jax.numpy as jnp
from jax.experimental import pallas as pl


def kernel(x, edge_index, edge_weight, W1, b1, W2, b2):
    raise NotImplementedError("write your pallas kernel here")


# speedup vs baseline: 6.0118x; 6.0118x over previous
import jax
import jax.numpy as jnp
from jax import lax
from jax.experimental import pallas as pl
from jax.experimental.pallas import tpu as pltpu
from jax.experimental.pallas import tpu_sc as plsc

N = 10000
D = 256
H = 128
N_SUB = 16
BATCH = 128
N_BATCH = 79
EDGES_PER_TILE = BATCH * N_BATCH
E_PAD = EDGES_PER_TILE * N_SUB
NP = 10240
ROWS_PER_TILE = NP // N_SUB
BR = 400

_MESH = plsc.VectorSubcoreMesh(core_axis_name="c", subcore_axis_name="s")


def _deg_body(dst_hbm, ew_hbm, deg_hbm, acc, idx_v, ew_v, zbuf):
    c = lax.axis_index("c")
    s = lax.axis_index("s")

    @pl.when(c == 0)
    def _():
        for i in range(640 // 16):
            zbuf[pl.ds(i * 16, 16)] = jnp.zeros((16,), jnp.float32)
        pltpu.sync_copy(zbuf, acc.at[pl.ds(s * 640, 640)])
        plsc.subcore_barrier()

        def body(g, carry):
            base = s * EDGES_PER_TILE + g * BATCH
            pltpu.sync_copy(dst_hbm.at[pl.ds(base, BATCH)], idx_v)
            pltpu.sync_copy(ew_hbm.at[pl.ds(base, BATCH)], ew_v)
            pltpu.sync_copy(ew_v, acc.at[idx_v], add=True)
            return carry

        lax.fori_loop(0, N_BATCH, body, 0)
        plsc.subcore_barrier()

        @pl.when(s == 0)
        def _():
            pltpu.sync_copy(acc, deg_hbm)


_deg_call = pl.kernel(
    _deg_body,
    out_type=jax.ShapeDtypeStruct((16 * 640,), jnp.float32),
    mesh=_MESH,
    scratch_types=[
        pltpu.VMEM_SHARED((16 * 640,), jnp.float32),
        pltpu.VMEM((BATCH,), jnp.int32),
        pltpu.VMEM((BATCH,), jnp.float32),
        pltpu.VMEM((640,), jnp.float32),
    ],
)


def _msg_body(h_hbm, src_hbm, dst_hbm, ew_hbm, agg_hbm,
              acc, idx_s, idx_d, ew_v, rows, zbuf, sem):
    c = lax.axis_index("c")
    s = lax.axis_index("s")

    for r in range(40):
        for k in range(8):
            zbuf[r, pl.ds(k * 16, 16)] = jnp.zeros((16,), jnp.float32)

    def zbody(i, carry):
        pltpu.sync_copy(zbuf, acc.at[pl.ds(s * ROWS_PER_TILE + i * 40, 40)])
        return carry

    lax.fori_loop(0, 16, zbody, 0)
    plsc.subcore_barrier()

    off = c * NP

    def body(g, carry):
        base = s * EDGES_PER_TILE + g * BATCH
        pltpu.sync_copy(src_hbm.at[pl.ds(base, BATCH)], idx_s)
        pltpu.sync_copy(dst_hbm.at[pl.ds(base, BATCH)], idx_d)
        pltpu.sync_copy(ew_hbm.at[pl.ds(base, BATCH)], ew_v)
        for i in range(BATCH // 16):
            idx_s[pl.ds(i * 16, 16)] = idx_s[pl.ds(i * 16, 16)] + off
        pltpu.async_copy(h_hbm.at[idx_s], rows, sem).wait()

        def scale(j, c2):
            wv = ew_v[pl.ds(j * 16, 16)]
            for l in range(16):
                w = wv[l]
                e = j * 16 + l
                for k in range(8):
                    rows[e, pl.ds(k * 16, 16)] = rows[e, pl.ds(k * 16, 16)] * w
            return c2

        lax.fori_loop(0, BATCH // 16, scale, 0)
        pltpu.sync_copy(rows, acc.at[idx_d], add=True)
        return carry

    lax.fori_loop(0, N_BATCH, body, 0)
    plsc.subcore_barrier()
    pltpu.sync_copy(acc.at[pl.ds(s * ROWS_PER_TILE, ROWS_PER_TILE)],
                    agg_hbm.at[pl.ds(off + s * ROWS_PER_TILE, ROWS_PER_TILE)])


_msg_call = pl.kernel(
    _msg_body,
    out_type=jax.ShapeDtypeStruct((2 * NP, H), jnp.float32),
    mesh=_MESH,
    scratch_types=[
        pltpu.VMEM_SHARED((NP, H), jnp.float32),
        pltpu.VMEM((BATCH,), jnp.int32),
        pltpu.VMEM((BATCH,), jnp.int32),
        pltpu.VMEM((BATCH,), jnp.float32),
        pltpu.VMEM((BATCH, H), jnp.float32),
        pltpu.VMEM((40, H), jnp.float32),
        pltpu.SemaphoreType.DMA,
    ],
)


def _mm1_body(x_ref, w_ref, deg_ref, h1p_ref, dinv_ref):
    dinv = lax.rsqrt(deg_ref[...] + 1.0)
    h = jnp.dot(x_ref[...], w_ref[...], preferred_element_type=jnp.float32)
    hp = h * dinv
    h1p_ref[0, :, :] = hp[:, :H]
    h1p_ref[1, :, :] = hp[:, H:]
    dinv_ref[...] = dinv


_mm1_call = pl.pallas_call(
    _mm1_body,
    grid=(N // BR,),
    in_specs=[
        pl.BlockSpec((BR, D), lambda i: (i, 0)),
        pl.BlockSpec((D, D), lambda i: (0, 0)),
        pl.BlockSpec((BR, 1), lambda i: (i, 0)),
    ],
    out_specs=[
        pl.BlockSpec((2, BR, H), lambda i: (0, i, 0)),
        pl.BlockSpec((BR, 1), lambda i: (i, 0)),
    ],
    out_shape=[
        jax.ShapeDtypeStruct((2, NP, H), jnp.float32),
        jax.ShapeDtypeStruct((N, 1), jnp.float32),
    ],
)


def _mid_body(agg_ref, hp_ref, dinv_ref, b_ref, w_ref, out_ref):
    dinv = dinv_ref[...]
    z = jnp.concatenate(
        [agg_ref[0, :, :] + hp_ref[0, :, :], agg_ref[1, :, :] + hp_ref[1, :, :]],
        axis=1)
    z = z * dinv + b_ref[...]
    z = jnp.where(z >= 0, z, 0.01 * z)
    h2 = jnp.dot(z, w_ref[...], preferred_element_type=jnp.float32)
    hp2 = h2 * dinv
    out_ref[0, :, :] = hp2[:, :H]
    out_ref[1, :, :] = hp2[:, H:]


_mid_call = pl.pallas_call(
    _mid_body,
    grid=(N // BR,),
    in_specs=[
        pl.BlockSpec((2, BR, H), lambda i: (0, i, 0)),
        pl.BlockSpec((2, BR, H), lambda i: (0, i, 0)),
        pl.BlockSpec((BR, 1), lambda i: (i, 0)),
        pl.BlockSpec((1, D), lambda i: (0, 0)),
        pl.BlockSpec((D, D), lambda i: (0, 0)),
    ],
    out_specs=pl.BlockSpec((2, BR, H), lambda i: (0, i, 0)),
    out_shape=jax.ShapeDtypeStruct((2, NP, H), jnp.float32),
)


def _out_body(agg_ref, hp_ref, dinv_ref, b_ref, out_ref):
    dinv = dinv_ref[...]
    z = jnp.concatenate(
        [agg_ref[0, :, :] + hp_ref[0, :, :], agg_ref[1, :, :] + hp_ref[1, :, :]],
        axis=1)
    z = z * dinv + b_ref[...]
    out_ref[...] = jnp.where(z >= 0, z, 0.01 * z)


_out_call = pl.pallas_call(
    _out_body,
    grid=(N // BR,),
    in_specs=[
        pl.BlockSpec((2, BR, H), lambda i: (0, i, 0)),
        pl.BlockSpec((2, BR, H), lambda i: (0, i, 0)),
        pl.BlockSpec((BR, 1), lambda i: (i, 0)),
        pl.BlockSpec((1, D), lambda i: (0, 0)),
    ],
    out_specs=pl.BlockSpec((BR, D), lambda i: (i, 0)),
    out_shape=jax.ShapeDtypeStruct((N, D), jnp.float32),
)


def kernel(x, edge_index, edge_weight, W1, b1, W2, b2):
    src = edge_index[0].astype(jnp.int32)
    dst = edge_index[1].astype(jnp.int32)
    pad = E_PAD - src.shape[0]
    src_p = jnp.concatenate([src, jnp.zeros((pad,), jnp.int32)])
    dst_p = jnp.concatenate([dst, jnp.zeros((pad,), jnp.int32)])
    ew_p = jnp.concatenate([edge_weight, jnp.zeros((pad,), jnp.float32)])

    deg = _deg_call(dst_p, ew_p)[:N]
    h1p, dinv = _mm1_call(x, W1, deg.reshape(N, 1))
    agg1 = _msg_call(h1p.reshape(2 * NP, H), src_p, dst_p, ew_p)
    h2p = _mid_call(agg1.reshape(2, NP, H), h1p, dinv, b1.reshape(1, D), W2)
    agg2 = _msg_call(h2p.reshape(2 * NP, H), src_p, dst_p, ew_p)
    return _out_call(agg2.reshape(2, NP, H), h2p, dinv, b2.reshape(1, D))


# speedup vs baseline: 7.8799x; 1.3107x over previous
import jax
import jax.numpy as jnp
from jax import lax
from jax.experimental import pallas as pl
from jax.experimental.pallas import tpu as pltpu
from jax.experimental.pallas import tpu_sc as plsc

N = 10000
D = 256
H = 128
N_SUB = 16
BATCH = 128
N_BATCH = 80
CHUNK = 16
EDGES_PER_TILE = BATCH * N_BATCH
E_PAD = EDGES_PER_TILE * N_SUB
NP = 10240
ROWS_PER_TILE = NP // N_SUB
BR = 400

_MESH = plsc.VectorSubcoreMesh(core_axis_name="c", subcore_axis_name="s")


def _deg_body(dst_hbm, ew_hbm, deg_hbm, acc, dst_sl, ew_sl, zbuf, sem):
    c = lax.axis_index("c")
    s = lax.axis_index("s")

    for i in range(640 // 16):
        zbuf[pl.ds(i * 16, 16)] = jnp.zeros((16,), jnp.float32)
    pltpu.sync_copy(zbuf, acc.at[pl.ds(s * 640, 640)])
    pltpu.sync_copy(dst_hbm.at[s], dst_sl)
    pltpu.sync_copy(ew_hbm.at[s], ew_sl)
    plsc.subcore_barrier()

    lo = c * 40

    def round_(r, carry):
        for k in range(8):
            b = lo + r * 8 + k
            pltpu.async_copy(ew_sl.at[b], acc.at[dst_sl.at[b]], sem, add=True)
        for k in range(8):
            pltpu.make_async_copy(ew_sl.at[lo], acc.at[dst_sl.at[lo]],
                                  sem).wait()
        return carry

    lax.fori_loop(0, 5, round_, 0)
    plsc.subcore_barrier()

    @pl.when(s == 0)
    def _():
        pltpu.sync_copy(acc, deg_hbm.at[c])


_deg_call = pl.kernel(
    _deg_body,
    out_type=jax.ShapeDtypeStruct((2, NP), jnp.float32),
    mesh=_MESH,
    scratch_types=[
        pltpu.VMEM_SHARED((NP,), jnp.float32),
        pltpu.VMEM((N_BATCH, BATCH), jnp.int32),
        pltpu.VMEM((N_BATCH, BATCH), jnp.float32),
        pltpu.VMEM((640,), jnp.float32),
        pltpu.SemaphoreType.DMA,
    ],
)


def _msg_body(h_hbm, src_hbm, dst_hbm, ew_hbm, agg_hbm,
              acc, src_ch, dst_ch, ew_ch, rows0, rows1, zbuf, sem0, sem1):
    c = lax.axis_index("c")
    s = lax.axis_index("s")

    for r in range(16):
        for k in range(8):
            zbuf[r, pl.ds(k * 16, 16)] = jnp.zeros((16,), jnp.float32)

    def zbody(i, carry):
        pltpu.sync_copy(zbuf, acc.at[pl.ds(s * ROWS_PER_TILE + i * 16, 16)])
        return carry

    lax.fori_loop(0, ROWS_PER_TILE // 16, zbody, 0)
    plsc.subcore_barrier()

    def start(g, buf, sem):
        pltpu.async_copy(h_hbm.at[src_ch.at[g]], buf, sem)

    def drain(buf, sem):
        pltpu.make_async_copy(h_hbm.at[src_ch.at[0]], buf, sem).wait()

    def process(g, buf):
        def scale(j, c2):
            wv = ew_ch[g, pl.ds(j * 16, 16)]
            for l in range(16):
                w = wv[l]
                e = j * 16 + l
                for k in range(8):
                    buf[e, pl.ds(k * 16, 16)] = buf[e, pl.ds(k * 16, 16)] * w
            return c2

        lax.fori_loop(0, BATCH // 16, scale, 0)
        pltpu.sync_copy(buf, acc.at[dst_ch.at[g]], add=True)

    def chunk(q, carry):
        pltpu.sync_copy(src_hbm.at[s, pl.ds(q * CHUNK, CHUNK)], src_ch)
        pltpu.sync_copy(dst_hbm.at[s, pl.ds(q * CHUNK, CHUNK)], dst_ch)
        pltpu.sync_copy(ew_hbm.at[s, pl.ds(q * CHUNK, CHUNK)], ew_ch)

        @pl.when(c == 1)
        def _():
            for g in range(CHUNK):
                for j in range(BATCH // 16):
                    src_ch[g, pl.ds(j * 16, 16)] = (
                        src_ch[g, pl.ds(j * 16, 16)] + NP)

        start(0, rows0, sem0)

        def body(t, carry2):
            g0 = 2 * t
            g1 = 2 * t + 1
            start(g1, rows1, sem1)
            drain(rows0, sem0)
            process(g0, rows0)

            @pl.when(g1 + 1 < CHUNK)
            def _():
                start(g1 + 1, rows0, sem0)

            drain(rows1, sem1)
            process(g1, rows1)
            return carry2

        lax.fori_loop(0, CHUNK // 2, body, 0)
        return carry

    lax.fori_loop(0, N_BATCH // CHUNK, chunk, 0)
    plsc.subcore_barrier()
    pltpu.sync_copy(acc.at[pl.ds(s * ROWS_PER_TILE, ROWS_PER_TILE)],
                    agg_hbm.at[pl.ds(c * NP + s * ROWS_PER_TILE,
                                     ROWS_PER_TILE)])


_msg_call = pl.kernel(
    _msg_body,
    out_type=jax.ShapeDtypeStruct((2 * NP, H), jnp.float32),
    mesh=_MESH,
    scratch_types=[
        pltpu.VMEM_SHARED((NP, H), jnp.float32),
        pltpu.VMEM((CHUNK, BATCH), jnp.int32),
        pltpu.VMEM((CHUNK, BATCH), jnp.int32),
        pltpu.VMEM((CHUNK, BATCH), jnp.float32),
        pltpu.VMEM((BATCH, H), jnp.float32),
        pltpu.VMEM((BATCH, H), jnp.float32),
        pltpu.VMEM((16, H), jnp.float32),
        pltpu.SemaphoreType.DMA,
        pltpu.SemaphoreType.DMA,
    ],
)


def _mm1_body(x_ref, w_ref, deg_ref, h1p_ref, dinv_ref):
    deg = 1.0 + deg_ref[:, 0:1] + deg_ref[:, 1:2]
    dinv = lax.rsqrt(deg)
    h = jnp.dot(x_ref[...], w_ref[...], preferred_element_type=jnp.float32)
    hp = h * dinv
    h1p_ref[0, :, :] = hp[:, :H]
    h1p_ref[1, :, :] = hp[:, H:]
    dinv_ref[...] = dinv


_mm1_call = pl.pallas_call(
    _mm1_body,
    grid=(N // BR,),
    in_specs=[
        pl.BlockSpec((BR, D), lambda i: (i, 0)),
        pl.BlockSpec((D, D), lambda i: (0, 0)),
        pl.BlockSpec((BR, 2), lambda i: (i, 0)),
    ],
    out_specs=[
        pl.BlockSpec((2, BR, H), lambda i: (0, i, 0)),
        pl.BlockSpec((BR, 1), lambda i: (i, 0)),
    ],
    out_shape=[
        jax.ShapeDtypeStruct((2, NP, H), jnp.float32),
        jax.ShapeDtypeStruct((N, 1), jnp.float32),
    ],
)


def _mid_body(agg_ref, hp_ref, dinv_ref, b_ref, w_ref, out_ref):
    dinv = dinv_ref[...]
    z = jnp.concatenate(
        [agg_ref[0, :, :] + hp_ref[0, :, :], agg_ref[1, :, :] + hp_ref[1, :, :]],
        axis=1)
    z = z * dinv + b_ref[...]
    z = jnp.where(z >= 0, z, 0.01 * z)
    h2 = jnp.dot(z, w_ref[...], preferred_element_type=jnp.float32)
    hp2 = h2 * dinv
    out_ref[0, :, :] = hp2[:, :H]
    out_ref[1, :, :] = hp2[:, H:]


_mid_call = pl.pallas_call(
    _mid_body,
    grid=(N // BR,),
    in_specs=[
        pl.BlockSpec((2, BR, H), lambda i: (0, i, 0)),
        pl.BlockSpec((2, BR, H), lambda i: (0, i, 0)),
        pl.BlockSpec((BR, 1), lambda i: (i, 0)),
        pl.BlockSpec((1, D), lambda i: (0, 0)),
        pl.BlockSpec((D, D), lambda i: (0, 0)),
    ],
    out_specs=pl.BlockSpec((2, BR, H), lambda i: (0, i, 0)),
    out_shape=jax.ShapeDtypeStruct((2, NP, H), jnp.float32),
)


def _out_body(agg_ref, hp_ref, dinv_ref, b_ref, out_ref):
    dinv = dinv_ref[...]
    z = jnp.concatenate(
        [agg_ref[0, :, :] + hp_ref[0, :, :], agg_ref[1, :, :] + hp_ref[1, :, :]],
        axis=1)
    z = z * dinv + b_ref[...]
    out_ref[...] = jnp.where(z >= 0, z, 0.01 * z)


_out_call = pl.pallas_call(
    _out_body,
    grid=(N // BR,),
    in_specs=[
        pl.BlockSpec((2, BR, H), lambda i: (0, i, 0)),
        pl.BlockSpec((2, BR, H), lambda i: (0, i, 0)),
        pl.BlockSpec((BR, 1), lambda i: (i, 0)),
        pl.BlockSpec((1, D), lambda i: (0, 0)),
    ],
    out_specs=pl.BlockSpec((BR, D), lambda i: (i, 0)),
    out_shape=jax.ShapeDtypeStruct((N, D), jnp.float32),
)


def kernel(x, edge_index, edge_weight, W1, b1, W2, b2):
    src = edge_index[0].astype(jnp.int32)
    dst = edge_index[1].astype(jnp.int32)
    pad = E_PAD - src.shape[0]
    src_p = jnp.concatenate([src, jnp.zeros((pad,), jnp.int32)])
    dst_p = jnp.concatenate([dst, jnp.zeros((pad,), jnp.int32)])
    ew_p = jnp.concatenate([edge_weight, jnp.zeros((pad,), jnp.float32)])
    src_sl = src_p.reshape(N_SUB, N_BATCH, BATCH)
    dst_sl = dst_p.reshape(N_SUB, N_BATCH, BATCH)
    ew_sl = ew_p.reshape(N_SUB, N_BATCH, BATCH)

    deg2 = _deg_call(dst_sl, ew_sl)
    degT = deg2.T[:N]
    h1p, dinv = _mm1_call(x, W1, degT)
    agg1 = _msg_call(h1p.reshape(2 * NP, H), src_sl, dst_sl, ew_sl)
    h2p = _mid_call(agg1.reshape(2, NP, H), h1p, dinv, b1.reshape(1, D), W2)
    agg2 = _msg_call(h2p.reshape(2 * NP, H), src_sl, dst_sl, ew_sl)
    return _out_call(agg2.reshape(2, NP, H), h2p, dinv, b2.reshape(1, D))
